# baseline (speedup 1.0000x reference)
import functools
import math

import jax
import jax.numpy as jnp
from jax import lax
from jax.experimental import pallas as pl
from jax.experimental.pallas import tpu as pltpu
from jax.experimental.pallas import tpu_sc as plsc

_THETA_FREQ = 6.0
_GAMMA_FREQ = 40.0
_D = 64

_NC = 2
_NS = 16
_NW = _NC * _NS


def _mod_table_body(t_ref, emb_ref, gp_ref, out_ref):
    t = t_ref[0, 0]
    gamma_t = 2.0 * math.pi * _GAMMA_FREQ * t
    scale = 0.5 + 0.5 * jnp.cos(gamma_t - gp_ref[:, :])
    out_ref[:, :] = emb_ref[:, :] * scale


def _modulated_table(embeddings, gamma_phases, t):
    n = embeddings.shape[0]
    t_arr = jnp.reshape(t, (1, 1)).astype(jnp.float32)
    gp2d = gamma_phases.reshape(n, 1)
    return pl.pallas_call(
        _mod_table_body,
        out_shape=jax.ShapeDtypeStruct((n, _D), jnp.float32),
        in_specs=[
            pl.BlockSpec(memory_space=pltpu.SMEM),
            pl.BlockSpec(memory_space=pltpu.VMEM),
            pl.BlockSpec(memory_space=pltpu.VMEM),
        ],
    )(t_arr, embeddings, gp2d)


def _sc_gather(flat_ids, table, n_tok, seq, chunk_t):
    n_rows = flat_ids.shape[0]
    per_w = n_rows // _NW
    tok_per_w = n_tok // _NW
    chunk = chunk_t * seq
    n_chunks = per_w // chunk
    assert n_chunks % 2 == 0
    seq_p = 56
    dp = 128
    mesh = plsc.VectorSubcoreMesh(core_axis_name="c", subcore_axis_name="s")

    @functools.partial(
        pl.kernel,
        out_type=jax.ShapeDtypeStruct((n_tok, seq_p, dp), jnp.float32),
        mesh=mesh,
        scratch_types=[
            pltpu.VMEM((per_w,), jnp.int32),
            pltpu.VMEM_SHARED((1000, _D), jnp.float32),
            pltpu.VMEM((chunk, _D), jnp.float32),
            pltpu.VMEM((chunk, _D), jnp.float32),
            pltpu.SemaphoreType.DMA,
            pltpu.SemaphoreType.DMA,
            pltpu.SemaphoreType.DMA,
        ],
        compiler_params=pltpu.CompilerParams(use_tc_tiling_on_sc=False),
    )
    def k(idx_hbm, table_hbm, out_hbm, idx_v, table_sh, buf0, buf1,
          gsem, ssem0, ssem1):
        wid = lax.axis_index("s") * _NC + lax.axis_index("c")
        w_base = wid * per_w
        w_tok = wid * tok_per_w

        @pl.when(lax.axis_index("s") == 0)
        def _():
            pltpu.sync_copy(table_hbm, table_sh)

        pltpu.sync_copy(idx_hbm.at[pl.ds(w_base, per_w)], idx_v)
        plsc.subcore_barrier()

        bufs = (buf0, buf1)
        ssems = (ssem0, ssem1)

        def store_chunk(buf, ssem, tok_base):
            for j in range(chunk_t):
                pltpu.async_copy(
                    buf.at[pl.ds(j * seq, seq)],
                    out_hbm.at[tok_base + j, pl.ds(0, seq), pl.ds(0, _D)],
                    ssem,
                )

        def drain_chunk(buf, ssem):
            for j in range(chunk_t):
                pltpu.make_async_copy(
                    buf.at[pl.ds(j * seq, seq)],
                    out_hbm.at[0, pl.ds(0, seq), pl.ds(0, _D)],
                    ssem,
                ).wait()

        def body(i2, carry):
            for b in range(2):
                i = i2 * 2 + b
                buf, ssem = bufs[b], ssems[b]
                @pl.when(i2 > 0)
                def _():
                    drain_chunk(buf, ssem)

                pltpu.async_copy(
                    table_sh.at[idx_v.at[pl.ds(i * chunk, chunk)]], buf, gsem
                ).wait()
                store_chunk(buf, ssem, w_tok + i * chunk_t)
            return carry

        lax.fori_loop(0, n_chunks // 2, body, 0)
        for b in range(2):
            drain_chunk(bufs[b], ssems[b])

    return k(flat_ids, table)


def kernel(concept_ids, embeddings, gamma_phases, t):
    table = _modulated_table(embeddings, gamma_phases, t)
    flat = concept_ids.reshape(-1).astype(jnp.int32)
    n_tok, seq = concept_ids.shape
    padded = _sc_gather(flat, table, n_tok, seq, chunk_t=8)
    theta_mod = 0.5 + 0.5 * jnp.cos(
        jnp.float32(2.0 * math.pi * _THETA_FREQ) * t.astype(jnp.float32)
    )
    return padded[:, :seq, :_D] * theta_mod


# speedup vs baseline: 1.9594x; 1.9594x over previous
import functools
import math

import jax
import jax.numpy as jnp
from jax import lax
from jax.experimental import pallas as pl
from jax.experimental.pallas import tpu as pltpu
from jax.experimental.pallas import tpu_sc as plsc

_THETA_FREQ = 6.0
_GAMMA_FREQ = 40.0
_D = 64

_NC = 2
_NS = 16
_NW = _NC * _NS


def _mod_table_body(t_ref, emb_ref, gp_ref, out_ref):
    t = t_ref[0, 0]
    theta_mod = 0.5 + 0.5 * jnp.cos(2.0 * math.pi * _THETA_FREQ * t)
    gamma_t = 2.0 * math.pi * _GAMMA_FREQ * t
    scale = theta_mod * (0.5 + 0.5 * jnp.cos(gamma_t - gp_ref[:, :]))
    out_ref[:, :] = emb_ref[:, :] * scale


def _modulated_table(embeddings, gamma_phases, t):
    n = embeddings.shape[0]
    t_arr = jnp.reshape(t, (1, 1)).astype(jnp.float32)
    gp2d = gamma_phases.reshape(n, 1)
    return pl.pallas_call(
        _mod_table_body,
        out_shape=jax.ShapeDtypeStruct((n, _D), jnp.float32),
        in_specs=[
            pl.BlockSpec(memory_space=pltpu.SMEM),
            pl.BlockSpec(memory_space=pltpu.VMEM),
            pl.BlockSpec(memory_space=pltpu.VMEM),
        ],
    )(t_arr, embeddings, gp2d)


def _sc_gather(flat_ids, table, n_tok, seq, chunk_t):
    n_rows = flat_ids.shape[0]
    per_w = n_rows // _NW
    tok_per_w = n_tok // _NW
    chunk = chunk_t * seq
    n_chunks = per_w // chunk
    assert n_chunks % 2 == 0
    seq_p = 56
    dp = 128
    mesh = plsc.VectorSubcoreMesh(core_axis_name="c", subcore_axis_name="s")

    @functools.partial(
        pl.kernel,
        out_type=jax.ShapeDtypeStruct((n_tok, seq_p, dp), jnp.float32),
        mesh=mesh,
        scratch_types=[
            pltpu.VMEM((per_w,), jnp.int32),
            pltpu.VMEM_SHARED((1000, _D), jnp.float32),
            pltpu.VMEM((chunk, _D), jnp.float32),
            pltpu.VMEM((chunk, _D), jnp.float32),
            pltpu.SemaphoreType.DMA,
            pltpu.SemaphoreType.DMA,
            pltpu.SemaphoreType.DMA,
        ],
        compiler_params=pltpu.CompilerParams(use_tc_tiling_on_sc=False),
    )
    def k(idx_hbm, table_hbm, out_hbm, idx_v, table_sh, buf0, buf1,
          gsem, ssem0, ssem1):
        wid = lax.axis_index("s") * _NC + lax.axis_index("c")
        w_base = wid * per_w
        w_tok = wid * tok_per_w

        @pl.when(lax.axis_index("s") == 0)
        def _():
            pltpu.sync_copy(table_hbm, table_sh)

        pltpu.sync_copy(idx_hbm.at[pl.ds(w_base, per_w)], idx_v)
        plsc.subcore_barrier()

        bufs = (buf0, buf1)
        ssems = (ssem0, ssem1)

        def store_chunk(buf, ssem, tok_base):
            for j in range(chunk_t):
                pltpu.async_copy(
                    buf.at[pl.ds(j * seq, seq)],
                    out_hbm.at[tok_base + j, pl.ds(0, seq), pl.ds(0, _D)],
                    ssem,
                )

        def drain_chunk(buf, ssem):
            for j in range(chunk_t):
                pltpu.make_async_copy(
                    buf.at[pl.ds(j * seq, seq)],
                    out_hbm.at[0, pl.ds(0, seq), pl.ds(0, _D)],
                    ssem,
                ).wait()

        def body(i2, carry):
            for b in range(2):
                i = i2 * 2 + b
                buf, ssem = bufs[b], ssems[b]
                @pl.when(i2 > 0)
                def _():
                    drain_chunk(buf, ssem)

                pltpu.async_copy(
                    table_sh.at[idx_v.at[pl.ds(i * chunk, chunk)]], buf, gsem
                ).wait()
                store_chunk(buf, ssem, w_tok + i * chunk_t)
            return carry

        lax.fori_loop(0, n_chunks // 2, body, 0)
        for b in range(2):
            drain_chunk(bufs[b], ssems[b])

    return k(flat_ids, table)


def kernel(concept_ids, embeddings, gamma_phases, t):
    table = _modulated_table(embeddings, gamma_phases, t)
    flat = concept_ids.reshape(-1).astype(jnp.int32)
    n_tok, seq = concept_ids.shape
    padded = _sc_gather(flat, table, n_tok, seq, chunk_t=8)
    return padded[:, :seq, :_D]


# speedup vs baseline: 1.9601x; 1.0004x over previous
import functools
import math

import jax
import jax.numpy as jnp
from jax import lax
from jax.experimental import pallas as pl
from jax.experimental.pallas import tpu as pltpu
from jax.experimental.pallas import tpu_sc as plsc

_THETA_FREQ = 6.0
_GAMMA_FREQ = 40.0
_D = 64

_NC = 2
_NS = 16
_NW = _NC * _NS


def _mod_table_body(t_ref, emb_ref, gp_ref, out_ref):
    t = t_ref[0, 0]
    theta_mod = 0.5 + 0.5 * jnp.cos(2.0 * math.pi * _THETA_FREQ * t)
    gamma_t = 2.0 * math.pi * _GAMMA_FREQ * t
    scale = theta_mod * (0.5 + 0.5 * jnp.cos(gamma_t - gp_ref[:, :]))
    out_ref[:, :] = emb_ref[:, :] * scale


def _modulated_table(embeddings, gamma_phases, t):
    n = embeddings.shape[0]
    t_arr = jnp.reshape(t, (1, 1)).astype(jnp.float32)
    gp2d = gamma_phases.reshape(n, 1)
    return pl.pallas_call(
        _mod_table_body,
        out_shape=jax.ShapeDtypeStruct((n, _D), jnp.float32),
        in_specs=[
            pl.BlockSpec(memory_space=pltpu.SMEM),
            pl.BlockSpec(memory_space=pltpu.VMEM),
            pl.BlockSpec(memory_space=pltpu.VMEM),
        ],
    )(t_arr, embeddings, gp2d)


def _sc_gather(flat_ids, table, n_tok, seq, chunk_t):
    n_rows = flat_ids.shape[0]
    per_w = n_rows // _NW
    tok_per_w = n_tok // _NW
    chunk = chunk_t * seq
    n_chunks = per_w // chunk
    assert n_chunks % 2 == 0
    seq_p = -(-seq // 8) * 8
    dp = 128
    mesh = plsc.VectorSubcoreMesh(core_axis_name="c", subcore_axis_name="s")

    @functools.partial(
        pl.kernel,
        out_type=jax.ShapeDtypeStruct((n_tok, seq_p, dp), jnp.float32),
        mesh=mesh,
        scratch_types=[
            pltpu.VMEM((per_w,), jnp.int32),
            pltpu.VMEM_SHARED(table.shape, jnp.float32),
            pltpu.VMEM((chunk, _D), jnp.float32),
            pltpu.VMEM((chunk, _D), jnp.float32),
            pltpu.SemaphoreType.DMA,
            pltpu.SemaphoreType.DMA,
            pltpu.SemaphoreType.DMA,
        ],
        compiler_params=pltpu.CompilerParams(use_tc_tiling_on_sc=False),
    )
    def k(idx_hbm, table_hbm, out_hbm, idx_v, table_sh, buf0, buf1,
          gsem, ssem0, ssem1):
        wid = lax.axis_index("s") * _NC + lax.axis_index("c")
        w_base = wid * per_w
        w_tok = wid * tok_per_w

        @pl.when(lax.axis_index("s") == 0)
        def _():
            pltpu.sync_copy(table_hbm, table_sh)

        pltpu.sync_copy(idx_hbm.at[pl.ds(w_base, per_w)], idx_v)
        plsc.subcore_barrier()

        bufs = (buf0, buf1)
        ssems = (ssem0, ssem1)

        def store_chunk(buf, ssem, tok_base):
            for j in range(chunk_t):
                pltpu.async_copy(
                    buf.at[pl.ds(j * seq, seq)],
                    out_hbm.at[tok_base + j, pl.ds(0, seq), pl.ds(0, _D)],
                    ssem,
                )

        def drain_chunk(buf, ssem):
            for j in range(chunk_t):
                pltpu.make_async_copy(
                    buf.at[pl.ds(j * seq, seq)],
                    out_hbm.at[0, pl.ds(0, seq), pl.ds(0, _D)],
                    ssem,
                ).wait()

        def body(i2, carry):
            for b in range(2):
                i = i2 * 2 + b
                buf, ssem = bufs[b], ssems[b]
                @pl.when(i2 > 0)
                def _():
                    drain_chunk(buf, ssem)

                pltpu.async_copy(
                    table_sh.at[idx_v.at[pl.ds(i * chunk, chunk)]], buf, gsem
                ).wait()
                store_chunk(buf, ssem, w_tok + i * chunk_t)
            return carry

        lax.fori_loop(0, n_chunks // 2, body, 0)
        for b in range(2):
            drain_chunk(bufs[b], ssems[b])

    return k(flat_ids, table)


def kernel(concept_ids, embeddings, gamma_phases, t):
    table = _modulated_table(embeddings, gamma_phases, t)
    flat = concept_ids.reshape(-1).astype(jnp.int32)
    n_tok, seq = concept_ids.shape
    padded = _sc_gather(flat, table, n_tok, seq, chunk_t=8)
    return padded[:, :seq, :_D]
